# baseline (speedup 1.0000x reference)
import dataclasses
import functools

import jax
import jax.numpy as jnp
from jax import lax
from jax.experimental import pallas as pl
from jax.experimental.pallas import tpu as pltpu
from jax.experimental.pallas import tpu_sc as plsc

C = 128
CAP = 512
D = 512
B = 8192
BLK = 2048
NSTEP = B // BLK
NZSTEP = (C * CAP) // BLK
FE_ROWS = B + BLK
ZROW = B

NW = 32
ROWS_PER_TILE = (C * CAP) // NW
G = 64
NCHUNK = ROWS_PER_TILE // G


def _prep_body(u_ref, label_ref, feat_ref, d_ref, len_ref, fe_ref, mem_ref,
               carry_ref):
    i = pl.program_id(0)

    mem_ref[...] = jnp.zeros_like(mem_ref)

    @pl.when(i <= NSTEP)
    def _():
        fe_ref[...] = jnp.where(i < NSTEP, feat_ref[...], 0.0)

    @pl.when(i == 0)
    def _():
        carry_ref[...] = jnp.zeros_like(carry_ref)

    @pl.when(i < NSTEP)
    def _():
        lb = label_ref[0, 0, :]
        cls = lax.broadcasted_iota(jnp.int32, (C, BLK), 0)
        onehot = cls == lb[None, :]
        csum = lax.dot_general(
            onehot.astype(jnp.bfloat16), u_ref[...],
            dimension_numbers=(((1,), (0,)), ((), ())),
            preferred_element_type=jnp.float32)
        total = csum + carry_ref[...]
        rank = jnp.sum(jnp.where(onehot, total, 0.0), axis=0) - 1.0
        rank_i = rank.astype(jnp.int32)
        dd = lb * CAP + rank_i
        dd = jnp.where(rank_i < CAP, dd, jnp.int32(2**30))
        d_ref[0, 0, :] = dd
        carry_ref[...] = carry_ref[...] + csum[:, BLK - 1:BLK]

    @pl.when(i == NSTEP - 1)
    def _():
        len_ref[...] = carry_ref[...].astype(jnp.int32)


_prep = pl.pallas_call(
    _prep_body,
    grid=(NZSTEP,),
    in_specs=[
        pl.BlockSpec((BLK, BLK), lambda i: (0, 0)),
        pl.BlockSpec((1, 1, BLK), lambda i: (jnp.minimum(i, NSTEP - 1), 0, 0)),
        pl.BlockSpec((BLK, D), lambda i: (jnp.minimum(i, NSTEP - 1), 0)),
    ],
    out_specs=[
        pl.BlockSpec((1, 1, BLK), lambda i: (jnp.minimum(i, NSTEP - 1), 0, 0)),
        pl.BlockSpec((C, 1), lambda i: (0, 0)),
        pl.BlockSpec((BLK, D), lambda i: (jnp.minimum(i, NSTEP), 0)),
        pl.BlockSpec((BLK, D), lambda i: (i, 0)),
    ],
    out_shape=[
        jax.ShapeDtypeStruct((NSTEP, 1, BLK), jnp.int32),
        jax.ShapeDtypeStruct((C, 1), jnp.int32),
        jax.ShapeDtypeStruct((FE_ROWS, D), jnp.float32),
        jax.ShapeDtypeStruct((C * CAP, D), jnp.float32),
    ],
    scratch_shapes=[pltpu.VMEM((C, 1), jnp.float32)],
)


def _sc_write_body(fe_hbm, d_hbm, mem_hbm, d_v, src_v, buf0, buf1,
                   g0, g1, w0, w1):
    wid = lax.axis_index("s") * 2 + lax.axis_index("c")
    base = wid * ROWS_PER_TILE
    pltpu.sync_copy(d_hbm, d_v)

    @pl.loop(0, ROWS_PER_TILE, step=16)
    def _(i):
        src_v[pl.ds(i, 16)] = lax.iota(jnp.int32, 16) + (i + ZROW)

    @pl.loop(0, B, step=16)
    def _(i):
        vd = d_v[pl.ds(i, 16)]
        loc = vd - base
        m = (loc >= 0) & (loc < ROWS_PER_TILE)
        locc = jnp.clip(loc, 0, ROWS_PER_TILE - 1)
        vi = lax.iota(jnp.int32, 16) + i
        plsc.store_scatter(src_v, [locc], vi, mask=m)

    @pl.loop(0, NCHUNK, step=2)
    def _(c):
        r0 = c * G
        r1 = r0 + G
        n0 = jnp.min(src_v[pl.ds(r0, 16)]) < B
        n1 = jnp.min(src_v[pl.ds(r1, 16)]) < B

        @pl.when(n0)
        def _():
            pltpu.async_copy(fe_hbm.at[src_v.at[pl.ds(r0, G)]], buf0, g0)

        @pl.when(n1)
        def _():
            pltpu.async_copy(fe_hbm.at[src_v.at[pl.ds(r1, G)]], buf1, g1)

        @pl.when(n0)
        def _():
            pltpu.make_async_copy(
                fe_hbm.at[src_v.at[pl.ds(r0, G)]], buf0, g0).wait()
            pltpu.async_copy(buf0, mem_hbm.at[pl.ds(base + r0, G)], w0)

        @pl.when(n1)
        def _():
            pltpu.make_async_copy(
                fe_hbm.at[src_v.at[pl.ds(r1, G)]], buf1, g1).wait()
            pltpu.async_copy(buf1, mem_hbm.at[pl.ds(base + r1, G)], w1)

        @pl.when(n0)
        def _():
            pltpu.make_async_copy(
                buf0, mem_hbm.at[pl.ds(base + r0, G)], w0).wait()

        @pl.when(n1)
        def _():
            pltpu.make_async_copy(
                buf1, mem_hbm.at[pl.ds(base + r1, G)], w1).wait()


@functools.cache
def _sc_write():
    mesh = plsc.VectorSubcoreMesh(core_axis_name="c", subcore_axis_name="s")
    cp = pltpu.CompilerParams()
    if "needs_layout_passes" in pltpu.CompilerParams.__dataclass_fields__:
        cp = dataclasses.replace(cp, needs_layout_passes=False)
    return pl.kernel(
        _sc_write_body,
        out_type=(),
        mesh=mesh,
        compiler_params=cp,
        scratch_types=[
            pltpu.VMEM((B,), jnp.int32),
            pltpu.VMEM((ROWS_PER_TILE,), jnp.int32),
            pltpu.VMEM((G, D), jnp.float32),
            pltpu.VMEM((G, D), jnp.float32),
            pltpu.SemaphoreType.DMA,
            pltpu.SemaphoreType.DMA,
            pltpu.SemaphoreType.DMA,
            pltpu.SemaphoreType.DMA,
        ],
    )


def kernel(feature, label):
    u = jnp.triu(jnp.ones((BLK, BLK), jnp.bfloat16))
    d3, mlen, fe, mem0 = _prep(u, label.reshape(NSTEP, 1, BLK), feature)
    mem_ref = jax.new_ref(mem0)
    _sc_write()(fe, d3.reshape(B), mem_ref)
    return mem_ref[...].reshape(C, CAP, D), mlen.reshape(C)


# speedup vs baseline: 1.0469x; 1.0469x over previous
import dataclasses
import functools

import jax
import jax.numpy as jnp
import numpy as np
from jax import lax
from jax.experimental import pallas as pl
from jax.experimental.pallas import tpu as pltpu
from jax.experimental.pallas import tpu_sc as plsc

C = 128
CAP = 512
D = 512
B = 8192
BLK = 2048
NSTEP = B // BLK
NZSTEP = (C * CAP) // BLK
FE_ROWS = B + BLK
ZROW = B

NW = 32
ROWS_PER_TILE = (C * CAP) // NW
NCLS_TILE = ROWS_PER_TILE // CAP
G = 64
NCHUNK = ROWS_PER_TILE // G

_U = np.triu(np.ones((BLK, BLK), np.float32)).astype(jnp.bfloat16)


def _prep_body(u_ref, label_ref, feat_ref, d_ref, len_ref, fe_ref, mem_ref,
               carry_ref):
    i = pl.program_id(0)

    mem_ref[...] = jnp.zeros_like(mem_ref)

    @pl.when(i <= NSTEP)
    def _():
        fe_ref[...] = jnp.where(i < NSTEP, feat_ref[...], 0.0)

    @pl.when(i == 0)
    def _():
        carry_ref[...] = jnp.zeros_like(carry_ref)

    @pl.when(i < NSTEP)
    def _():
        lb = label_ref[0, 0, :]
        cls = lax.broadcasted_iota(jnp.int32, (C, BLK), 0)
        onehot = cls == lb[None, :]
        csum = lax.dot_general(
            onehot.astype(jnp.bfloat16), u_ref[...],
            dimension_numbers=(((1,), (0,)), ((), ())),
            preferred_element_type=jnp.float32)
        total = csum + carry_ref[...]
        rank = jnp.sum(jnp.where(onehot, total, 0.0), axis=0) - 1.0
        rank_i = rank.astype(jnp.int32)
        dd = lb * CAP + rank_i
        dd = jnp.where(rank_i < CAP, dd, jnp.int32(2**30))
        d_ref[0, 0, :] = dd
        carry_ref[...] = carry_ref[...] + csum[:, BLK - 1:BLK]

    @pl.when(i == NSTEP - 1)
    def _():
        len_ref[...] = carry_ref[...].astype(jnp.int32)


_prep = pl.pallas_call(
    _prep_body,
    grid=(NZSTEP,),
    in_specs=[
        pl.BlockSpec((BLK, BLK), lambda i: (0, 0)),
        pl.BlockSpec((1, 1, BLK), lambda i: (jnp.minimum(i, NSTEP - 1), 0, 0)),
        pl.BlockSpec((BLK, D), lambda i: (jnp.minimum(i, NSTEP - 1), 0)),
    ],
    out_specs=[
        pl.BlockSpec((1, 1, BLK), lambda i: (jnp.minimum(i, NSTEP - 1), 0, 0)),
        pl.BlockSpec((C, 1), lambda i: (0, 0)),
        pl.BlockSpec((BLK, D), lambda i: (jnp.minimum(i, NSTEP), 0)),
        pl.BlockSpec((BLK, D), lambda i: (i, 0)),
    ],
    out_shape=[
        jax.ShapeDtypeStruct((NSTEP, 1, BLK), jnp.int32),
        jax.ShapeDtypeStruct((C, 1), jnp.int32),
        jax.ShapeDtypeStruct((FE_ROWS, D), jnp.float32),
        jax.ShapeDtypeStruct((C * CAP, D), jnp.float32),
    ],
    scratch_shapes=[pltpu.VMEM((C, 1), jnp.float32)],
)


def _sc_write_body(fe_hbm, d_hbm, mem_hbm, d_v, src_v, buf0, buf1,
                   g0, g1, w0, w1):
    wid = lax.axis_index("s") * 2 + lax.axis_index("c")
    base = wid * ROWS_PER_TILE
    pltpu.sync_copy(d_hbm, d_v)

    @pl.loop(0, ROWS_PER_TILE, step=16, unroll=8)
    def _(i):
        src_v[pl.ds(i, 16)] = lax.iota(jnp.int32, 16) + (i + ZROW)

    @pl.loop(0, B, step=16, unroll=4)
    def _(i):
        vd = d_v[pl.ds(i, 16)]
        loc = vd - base
        m = (loc >= 0) & (loc < ROWS_PER_TILE)
        locc = jnp.clip(loc, 0, ROWS_PER_TILE - 1)
        vi = lax.iota(jnp.int32, 16) + i
        plsc.store_scatter(src_v, [locc], vi, mask=m)

    lanes = lax.iota(jnp.int32, 16)
    for cls in range(NCLS_TILE):
        cbase = cls * CAP
        fidx = jnp.minimum(lanes * G + cbase, ROWS_PER_TILE - 1)
        firsts = plsc.load_gather(src_v, [fidx])
        nact = (firsts < B) & (lanes < CAP // G)
        t = jnp.sum(nact.astype(jnp.int32))

        def _pair(j, _, cbase=cbase, t=t):
            r0 = cbase + (2 * j) * G
            r1 = r0 + G
            more = 2 * j + 1 < t
            pltpu.async_copy(fe_hbm.at[src_v.at[pl.ds(r0, G)]], buf0, g0)

            @pl.when(more)
            def _():
                pltpu.async_copy(fe_hbm.at[src_v.at[pl.ds(r1, G)]], buf1, g1)

            pltpu.make_async_copy(
                fe_hbm.at[src_v.at[pl.ds(r0, G)]], buf0, g0).wait()
            pltpu.async_copy(buf0, mem_hbm.at[pl.ds(base + r0, G)], w0)

            @pl.when(more)
            def _():
                pltpu.make_async_copy(
                    fe_hbm.at[src_v.at[pl.ds(r1, G)]], buf1, g1).wait()
                pltpu.async_copy(buf1, mem_hbm.at[pl.ds(base + r1, G)], w1)

            pltpu.make_async_copy(
                buf0, mem_hbm.at[pl.ds(base + r0, G)], w0).wait()

            @pl.when(more)
            def _():
                pltpu.make_async_copy(
                    buf1, mem_hbm.at[pl.ds(base + r1, G)], w1).wait()

            return 0

        lax.fori_loop(0, (t + 1) // 2, _pair, 0)


@functools.cache
def _sc_write():
    mesh = plsc.VectorSubcoreMesh(core_axis_name="c", subcore_axis_name="s")
    cp = pltpu.CompilerParams()
    if "needs_layout_passes" in pltpu.CompilerParams.__dataclass_fields__:
        cp = dataclasses.replace(cp, needs_layout_passes=False)
    return pl.kernel(
        _sc_write_body,
        out_type=(),
        mesh=mesh,
        compiler_params=cp,
        scratch_types=[
            pltpu.VMEM((B,), jnp.int32),
            pltpu.VMEM((ROWS_PER_TILE,), jnp.int32),
            pltpu.VMEM((G, D), jnp.float32),
            pltpu.VMEM((G, D), jnp.float32),
            pltpu.SemaphoreType.DMA,
            pltpu.SemaphoreType.DMA,
            pltpu.SemaphoreType.DMA,
            pltpu.SemaphoreType.DMA,
        ],
    )


def kernel(feature, label):
    u = jnp.asarray(_U)
    d3, mlen, fe, mem0 = _prep(u, label.reshape(NSTEP, 1, BLK), feature)
    mem_ref = jax.new_ref(mem0)
    _sc_write()(fe, d3.reshape(B), mem_ref)
    return mem_ref[...].reshape(C, CAP, D), mlen.reshape(C)
